# Initial kernel scaffold; baseline (speedup 1.0000x reference)
import jax
import jax.numpy as jnp
from jax.experimental import pallas as pl


def kernel(x, edge_index, W1, b1, W2, b2, W_out, b_out):
    raise NotImplementedError("write your pallas kernel here")


# speedup vs baseline: 10.5021x; 10.5021x over previous
import functools

import jax
import jax.numpy as jnp
from jax import lax
from jax.experimental import pallas as pl
from jax.experimental.pallas import tpu as pltpu
from jax.experimental.pallas import tpu_sc as plsc

NC = 2
NS = 16
NW = NC * NS
K = 128
BM = 1024


def _sc_mesh():
    return plsc.VectorSubcoreMesh(core_axis_name="c", subcore_axis_name="s",
                                  num_cores=NC, num_subcores=NS)


def _make_deg_kernel(npad, ept):
    @functools.partial(
        pl.kernel,
        out_type=jax.ShapeDtypeStruct((NC, npad, 16), jnp.float32),
        mesh=_sc_mesh(),
        scratch_types=[
            pltpu.VMEM((K,), jnp.int32),
            pltpu.VMEM((K, 16), jnp.float32),
            pltpu.VMEM_SHARED((npad, 16), jnp.float32),
        ],
    )
    def deg_kernel(dst_hbm, ones_hbm, z16_hbm, out_hbm, dst_v, ones_v, acc_sh):
        cid = lax.axis_index("c")
        sid = lax.axis_index("s")
        wid = sid * NC + cid
        zrows = npad // NS
        pltpu.sync_copy(z16_hbm.at[pl.ds(sid * zrows, zrows)],
                        acc_sh.at[pl.ds(sid * zrows, zrows)])
        pltpu.sync_copy(ones_hbm, ones_v)
        plsc.subcore_barrier()
        base = wid * ept

        @pl.loop(0, ept, step=K)
        def _(e0):
            pltpu.sync_copy(dst_hbm.at[pl.ds(base + e0, K)], dst_v)
            pltpu.sync_copy(ones_v, acc_sh.at[dst_v], add=True)

        plsc.subcore_barrier()
        pltpu.sync_copy(acc_sh.at[pl.ds(sid * zrows, zrows)],
                        out_hbm.at[cid, pl.ds(sid * zrows, zrows)])

    return deg_kernel


def _make_agg_kernel(npad, d, ept):
    @functools.partial(
        pl.kernel,
        out_type=jax.ShapeDtypeStruct((NC, npad, d), jnp.float32),
        mesh=_sc_mesh(),
        scratch_types=[
            pltpu.VMEM((K,), jnp.int32),
            pltpu.VMEM((K,), jnp.int32),
            pltpu.VMEM((K, d), jnp.float32),
            pltpu.VMEM_SHARED((npad, d), jnp.float32),
            pltpu.SemaphoreType.DMA,
        ],
    )
    def agg_kernel(hs_hbm, src_hbm, dst_hbm, zrow_hbm, out_hbm,
                   src_v, dst_v, rows_v, acc_sh, sem):
        cid = lax.axis_index("c")
        sid = lax.axis_index("s")
        wid = sid * NC + cid
        zrows = npad // NS
        pltpu.sync_copy(zrow_hbm.at[pl.ds(sid * zrows, zrows)],
                        acc_sh.at[pl.ds(sid * zrows, zrows)])
        plsc.subcore_barrier()
        base = wid * ept

        @pl.loop(0, ept, step=K)
        def _(e0):
            pltpu.sync_copy(src_hbm.at[pl.ds(base + e0, K)], src_v)
            pltpu.sync_copy(dst_hbm.at[pl.ds(base + e0, K)], dst_v)
            pltpu.async_copy(hs_hbm.at[src_v], rows_v, sem).wait()
            pltpu.sync_copy(rows_v, acc_sh.at[dst_v], add=True)

        plsc.subcore_barrier()
        pltpu.sync_copy(acc_sh.at[pl.ds(sid * zrows, zrows)],
                        out_hbm.at[cid, pl.ds(sid * zrows, zrows)])

    return agg_kernel


def _matmul(x, w):
    m, k = x.shape
    n = w.shape[1]

    def body(x_ref, w_ref, o_ref):
        o_ref[...] = jnp.dot(x_ref[...], w_ref[...],
                             preferred_element_type=jnp.float32)

    return pl.pallas_call(
        body,
        grid=(m // BM,),
        in_specs=[pl.BlockSpec((BM, k), lambda i: (i, 0)),
                  pl.BlockSpec((k, n), lambda i: (0, 0))],
        out_specs=pl.BlockSpec((BM, n), lambda i: (i, 0)),
        out_shape=jax.ShapeDtypeStruct((m, n), jnp.float32),
    )(x, w)


def _dis_hs(degp, h):
    npad, d = h.shape

    def body(degp_ref, h_ref, dis_ref, hs_ref):
        deg = degp_ref[0] + degp_ref[1] + 1.0
        dis = lax.rsqrt(deg)
        dis_ref[...] = dis
        hs_ref[...] = h_ref[...] * dis[:, :1]

    return pl.pallas_call(
        body,
        grid=(npad // BM,),
        in_specs=[pl.BlockSpec((NC, BM, 16), lambda i: (0, i, 0)),
                  pl.BlockSpec((BM, d), lambda i: (i, 0))],
        out_specs=[pl.BlockSpec((BM, 16), lambda i: (i, 0)),
                   pl.BlockSpec((BM, d), lambda i: (i, 0))],
        out_shape=[jax.ShapeDtypeStruct((npad, 16), jnp.float32),
                   jax.ShapeDtypeStruct((npad, d), jnp.float32)],
    )(degp, h)


def _layer_mid(accp, h, dis, b, w):
    npad, d = h.shape
    n2 = w.shape[1]

    def body(accp_ref, h_ref, dis_ref, b_ref, w_ref, h2_ref, hs2_ref):
        dis_c = dis_ref[...][:, :1]
        acc = accp_ref[0] + accp_ref[1]
        out1 = dis_c * acc + (dis_c * dis_c) * h_ref[...] + b_ref[...]
        x2 = jnp.maximum(out1, 0.0)
        h2 = jnp.dot(x2, w_ref[...], preferred_element_type=jnp.float32)
        h2_ref[...] = h2
        hs2_ref[...] = h2 * dis_c

    return pl.pallas_call(
        body,
        grid=(npad // BM,),
        in_specs=[pl.BlockSpec((NC, BM, d), lambda i: (0, i, 0)),
                  pl.BlockSpec((BM, d), lambda i: (i, 0)),
                  pl.BlockSpec((BM, 16), lambda i: (i, 0)),
                  pl.BlockSpec((1, d), lambda i: (0, 0)),
                  pl.BlockSpec((d, n2), lambda i: (0, 0))],
        out_specs=[pl.BlockSpec((BM, n2), lambda i: (i, 0)),
                   pl.BlockSpec((BM, n2), lambda i: (i, 0))],
        out_shape=[jax.ShapeDtypeStruct((npad, n2), jnp.float32),
                   jax.ShapeDtypeStruct((npad, n2), jnp.float32)],
    )(accp, h, dis, b, w)


def _layer_out(accp, h, dis, b, w_out, b_out):
    npad, d = h.shape
    n_out = w_out.shape[1]

    def body(accp_ref, h_ref, dis_ref, b_ref, w_ref, bo_ref, o_ref):
        dis_c = dis_ref[...][:, :1]
        acc = accp_ref[0] + accp_ref[1]
        out2 = dis_c * acc + (dis_c * dis_c) * h_ref[...] + b_ref[...]
        x3 = jnp.maximum(out2, 0.0)
        o_ref[...] = jnp.dot(x3, w_ref[...],
                             preferred_element_type=jnp.float32) + bo_ref[...]

    return pl.pallas_call(
        body,
        grid=(npad // BM,),
        in_specs=[pl.BlockSpec((NC, BM, d), lambda i: (0, i, 0)),
                  pl.BlockSpec((BM, d), lambda i: (i, 0)),
                  pl.BlockSpec((BM, 16), lambda i: (i, 0)),
                  pl.BlockSpec((1, d), lambda i: (0, 0)),
                  pl.BlockSpec((d, n_out), lambda i: (0, 0)),
                  pl.BlockSpec((1, n_out), lambda i: (0, 0))],
        out_specs=pl.BlockSpec((BM, n_out), lambda i: (i, 0)),
        out_shape=jax.ShapeDtypeStruct((npad, n_out), jnp.float32),
    )(accp, h, dis, b, w_out, b_out)


def kernel(x, edge_index, W1, b1, W2, b2, W_out, b_out):
    n, d = x.shape
    e = edge_index.shape[1]
    npad = ((n + 1 + BM - 1) // BM) * BM
    epad = ((e + NW * K - 1) // (NW * K)) * (NW * K)
    ept = epad // NW

    x_pad = jnp.pad(x, ((0, npad - n), (0, 0)))
    src = jnp.concatenate(
        [edge_index[0], jnp.zeros((epad - e,), jnp.int32)])
    dst = jnp.concatenate(
        [edge_index[1], jnp.full((epad - e,), n, jnp.int32)])
    ones16 = jnp.ones((K, 16), jnp.float32)
    z16 = jnp.zeros((npad, 16), jnp.float32)
    zrow = jnp.zeros((npad, d), jnp.float32)

    deg_k = _make_deg_kernel(npad, ept)
    agg_k = _make_agg_kernel(npad, d, ept)

    degp = deg_k(dst, ones16, z16)
    h1 = _matmul(x_pad, W1)
    dis, hs1 = _dis_hs(degp, h1)
    acc1 = agg_k(hs1, src, dst, zrow)
    h2, hs2 = _layer_mid(acc1, h1, dis, b1[None, :], W2)
    acc2 = agg_k(hs2, src, dst, zrow)
    out = _layer_out(acc2, h2, dis, b2[None, :], W_out, b_out[None, :])
    return out[:n]
